# baseline (speedup 1.0000x reference)
import functools

import jax
import jax.numpy as jnp
from jax import lax
from jax.experimental import pallas as pl
from jax.experimental.pallas import tpu as pltpu
from jax.experimental.pallas import tpu_sc as plsc

NC, NS, L = 2, 16, 16
NW = NC * NS
B = 16384
D = 32
BPW = B // NW
CH = 128
NCH = BPW // CH
EPS = 1e-05


def _sc_body(left_hbm, right_hbm, w_hbm, uu_hbm, uv_hbm, vv_hbm,
             li_v, ri_v, u_v, v_v, uu_v, uv_v, vv_v, sem):
    wid = lax.axis_index("s") * NC + lax.axis_index("c")
    pltpu.sync_copy(left_hbm.at[wid], li_v)
    pltpu.sync_copy(right_hbm.at[wid], ri_v)
    copies = []
    for j in range(NCH):
        copies.append(pltpu.async_copy(
            w_hbm.at[li_v.at[j]], u_v.at[pl.ds(j * CH, CH)], sem))
        copies.append(pltpu.async_copy(
            w_hbm.at[ri_v.at[j]], v_v.at[pl.ds(j * CH, CH)], sem))
    for c in copies:
        c.wait()

    lane = lax.iota(jnp.int32, L)

    def group(g, carry):
        r0 = g * L
        uu = jnp.zeros((L,), jnp.float32)
        vv = jnp.zeros((L,), jnp.float32)
        uv = jnp.zeros((L,), jnp.float32)
        for k in range(L):
            u0 = u_v[r0 + k, pl.ds(0, L)]
            u1 = u_v[r0 + k, pl.ds(L, L)]
            v0 = v_v[r0 + k, pl.ds(0, L)]
            v1 = v_v[r0 + k, pl.ds(L, L)]
            m = lane == k
            uu = jnp.where(m, jnp.sum(u0 * u0 + u1 * u1), uu)
            vv = jnp.where(m, jnp.sum(v0 * v0 + v1 * v1), vv)
            uv = jnp.where(m, jnp.sum(u0 * v0 + u1 * v1), uv)
        uu_v[pl.ds(r0, L)] = uu
        vv_v[pl.ds(r0, L)] = vv
        uv_v[pl.ds(r0, L)] = uv
        return carry

    lax.fori_loop(0, BPW // L, group, 0)
    pltpu.sync_copy(uu_v, uu_hbm.at[wid])
    pltpu.sync_copy(uv_v, uv_hbm.at[wid])
    pltpu.sync_copy(vv_v, vv_hbm.at[wid])


_sc_call = pl.kernel(
    _sc_body,
    out_type=[jax.ShapeDtypeStruct((NW, BPW), jnp.float32)] * 3,
    mesh=plsc.VectorSubcoreMesh(
        core_axis_name="c", subcore_axis_name="s",
        num_cores=NC, num_subcores=NS),
    compiler_params=pltpu.CompilerParams(
        needs_layout_passes=False, use_tc_tiling_on_sc=False),
    scratch_types=[
        pltpu.VMEM((NCH, CH), jnp.int32),
        pltpu.VMEM((NCH, CH), jnp.int32),
        pltpu.VMEM((BPW, D), jnp.float32),
        pltpu.VMEM((BPW, D), jnp.float32),
        pltpu.VMEM((BPW,), jnp.float32),
        pltpu.VMEM((BPW,), jnp.float32),
        pltpu.VMEM((BPW,), jnp.float32),
        pltpu.SemaphoreType.DMA,
    ],
)


def _tc_body(uu_ref, uv_ref, vv_ref, alpha_ref, beta_ref, gamma_ref, dists_ref):
    uu = uu_ref[...]
    uv = uv_ref[...]
    vv = vv_ref[...]
    alpha = 1.0 - uu
    alpha = jnp.where(alpha <= 0.0, EPS, alpha)
    beta = 1.0 - vv
    beta = jnp.where(beta <= 0.0, EPS, beta)
    gamma = 1.0 + 2.0 * (uu - 2.0 * uv + vv) / alpha / beta
    gamma = jnp.where(gamma < 1.0, 1.0, gamma)
    alpha_ref[...] = alpha
    beta_ref[...] = beta
    gamma_ref[...] = gamma
    dists_ref[...] = jnp.log(gamma + jnp.sqrt(gamma * gamma - 1.0))


_tc_call = pl.pallas_call(
    _tc_body,
    out_shape=[jax.ShapeDtypeStruct((B // 128, 128), jnp.float32)] * 4,
)


def kernel(left_idx, right_idx, W):
    li = left_idx.reshape(NW, NCH, CH)
    ri = right_idx.reshape(NW, NCH, CH)
    uu, uv, vv = _sc_call(li, ri, W)
    uu2 = uu.reshape(B // 128, 128)
    uv2 = uv.reshape(B // 128, 128)
    vv2 = vv.reshape(B // 128, 128)
    alpha, beta, gamma, dists = _tc_call(uu2, uv2, vv2)
    return (uu.reshape(B), uv.reshape(B), vv.reshape(B),
            alpha.reshape(B), beta.reshape(B), gamma.reshape(B),
            dists.reshape(B))


# speedup vs baseline: 2.0923x; 2.0923x over previous
import jax
import jax.numpy as jnp
from jax import lax
from jax.experimental import pallas as pl
from jax.experimental.pallas import tpu as pltpu
from jax.experimental.pallas import tpu_sc as plsc

NC, NS, L = 2, 16, 16
NW = NC * NS
B = 16384
D = 32
V = 1000000
WIN = 512
NWIN = 1954
TAILW = 64
KMAX = 62
NCHUNK = B // L
STAGE_ROWS = 2 * B + NW
EPS = 1e-05


def _sc_body(wt_hbm, left_hbm, right_hbm, stage_hbm,
             idx_v, my_key, my_val, wbuf, ext, p_c, p_s,
             cnt_s, sctr_s, dsem0, dsem1, ssem):
    wid = lax.axis_index("s") * NC + lax.axis_index("c")
    lane = lax.iota(jnp.int32, L)
    dlo = lax.iota(jnp.int32, L)
    dhi = dlo + L

    def win_start(k):
        return (wid + k * NW) * WIN

    def issue_window(k):
        wb = wid + k * NW
        s = win_start(k)
        for par in range(2):
            @pl.when((wb < NWIN) & (k & 1 == par))
            def _():
                sem = dsem0 if par == 0 else dsem1

                @pl.when(wb < NWIN - 1)
                def _():
                    for db in range(4):
                        pltpu.async_copy(
                            wt_hbm.at[pl.ds(db * 8, 8), pl.ds(s, WIN)],
                            wbuf.at[par, pl.ds(db * 8, 8), :], sem)

                @pl.when(wb == NWIN - 1)
                def _():
                    for db in range(4):
                        pltpu.async_copy(
                            wt_hbm.at[pl.ds(db * 8, 8), pl.ds(s, TAILW)],
                            wbuf.at[par, pl.ds(db * 8, 8), pl.ds(0, TAILW)],
                            sem)

    def wait_window(k):
        wb = wid + k * NW
        s = win_start(k)
        for par in range(2):
            @pl.when((wb < NWIN) & (k & 1 == par))
            def _():
                sem = dsem0 if par == 0 else dsem1

                @pl.when(wb < NWIN - 1)
                def _():
                    for db in range(4):
                        pltpu.make_async_copy(
                            wt_hbm.at[pl.ds(db * 8, 8), pl.ds(s, WIN)],
                            wbuf.at[par, pl.ds(db * 8, 8), :], sem).wait()

                @pl.when(wb == NWIN - 1)
                def _():
                    for db in range(4):
                        pltpu.make_async_copy(
                            wt_hbm.at[pl.ds(db * 8, 8), pl.ds(s, TAILW)],
                            wbuf.at[par, pl.ds(db * 8, 8), pl.ds(0, TAILW)],
                            sem).wait()

    def drain_scatter(n):
        def body(_, c):
            pltpu.make_async_copy(ext.at[0], stage_hbm.at[pl.ds(0, L)],
                                  ssem).wait()
            return c
        lax.fori_loop(0, n, body, 0)

    def extract_group(par, cvec, slots):
        g = sctr_s[0]
        epar = lax.rem(g, 4)

        @pl.when(g >= 4)
        def _():
            pltpu.make_async_copy(ext.at[0], stage_hbm.at[pl.ds(0, L)],
                                  ssem).wait()
        for e in range(L):
            c_e = cvec[e]
            col = jnp.full((L,), 0, jnp.int32) + c_e
            lo = plsc.load_gather(wbuf.at[par], [dlo, col])
            hi = plsc.load_gather(wbuf.at[par], [dhi, col])
            ext[epar, e, pl.ds(0, L)] = lo
            ext[epar, e, pl.ds(L, L)] = hi
        pltpu.async_copy(ext.at[epar], stage_hbm.at[slots], ssem)
        sctr_s[0] = g + 1

    def process_window(k):
        wb = wid + k * NW
        cnt = cnt_s[0]
        for par in range(2):
            @pl.when((wb < NWIN) & (k & 1 == par))
            def _():
                def chunk_body(i, pcnt):
                    keys = my_key[pl.ds(i * L, L)]
                    vals = my_val[pl.ds(i * L, L)]
                    valid = (i * L + lane) < cnt
                    m = ((keys >> 9) == wb) & valid
                    pop = plsc.all_reduce_population_count(m)[0]
                    plsc.store_compressed(p_c.at[pl.ds(pcnt, L)],
                                          keys & 511, mask=m)
                    plsc.store_compressed(p_s.at[pl.ds(pcnt, L)], vals, mask=m)
                    pcnt = pcnt + pop

                    @pl.when(pcnt >= L)
                    def _():
                        extract_group(par, p_c[pl.ds(0, L)], p_s[pl.ds(0, L)])
                        t1 = p_c[pl.ds(L, L)]
                        p_c[pl.ds(0, L)] = t1
                        t2 = p_s[pl.ds(L, L)]
                        p_s[pl.ds(0, L)] = t2
                    return jnp.where(pcnt >= L, pcnt - L, pcnt)

                trip = (cnt + L - 1) // L
                pcnt = lax.fori_loop(0, trip, chunk_body, 0)

                @pl.when(pcnt > 0)
                def _():
                    cvec = p_c[pl.ds(0, L)]
                    svec = p_s[pl.ds(0, L)]
                    slots = jnp.where(lane < pcnt, svec, 2 * B + wid)
                    cvec = jnp.where(lane < pcnt, cvec, 0)
                    extract_group(par, cvec, slots)

    issue_window(0)

    def scan_side(side, idx_hbm, cnt0):
        pltpu.sync_copy(idx_hbm, idx_v)

        def scan_chunk(i, cnt):
            ks = idx_v[pl.ds(i * L, L)]
            mine = ((ks >> 9) & (NW - 1)) == wid
            slots = i * L + lane + side * B
            pop = plsc.all_reduce_population_count(mine)[0]
            plsc.store_compressed(my_key.at[pl.ds(cnt, L)], ks, mask=mine)
            plsc.store_compressed(my_val.at[pl.ds(cnt, L)], slots, mask=mine)
            return cnt + pop

        return lax.fori_loop(0, NCHUNK, scan_chunk, cnt0)

    cnt = scan_side(0, left_hbm, 0)
    cnt = scan_side(1, right_hbm, cnt)
    cnt_s[0] = cnt
    sctr_s[0] = 0

    def wloop(k, carry):
        issue_window(k + 1)
        wait_window(k)
        process_window(k)
        return carry

    lax.fori_loop(0, KMAX, wloop, 0)

    g = sctr_s[0]
    drain_scatter(jnp.minimum(g, 4))


_sc_call = pl.kernel(
    _sc_body,
    out_type=jax.ShapeDtypeStruct((STAGE_ROWS, 128), jnp.float32),
    mesh=plsc.VectorSubcoreMesh(
        core_axis_name="c", subcore_axis_name="s",
        num_cores=NC, num_subcores=NS),
    compiler_params=pltpu.CompilerParams(
        needs_layout_passes=False, use_tc_tiling_on_sc=True),
    scratch_types=[
        pltpu.VMEM((B,), jnp.int32),
        pltpu.VMEM((2 * B,), jnp.int32),
        pltpu.VMEM((2 * B,), jnp.int32),
        pltpu.VMEM((2, D, WIN), jnp.float32),
        pltpu.VMEM((4, L, 128), jnp.float32),
        pltpu.VMEM((2 * L,), jnp.int32),
        pltpu.VMEM((2 * L,), jnp.int32),
        pltpu.SMEM((1,), jnp.int32),
        pltpu.SMEM((1,), jnp.int32),
        pltpu.SemaphoreType.DMA,
        pltpu.SemaphoreType.DMA,
        pltpu.SemaphoreType.DMA,
    ],
)


TCB = 512


def _tc_body(u_ref, v_ref, uu_ref, uv_ref, vv_ref,
             alpha_ref, beta_ref, gamma_ref, dists_ref):
    u = u_ref[:, 0:D]
    v = v_ref[:, 0:D]
    uu = jnp.sum(u * u, axis=1)
    vv = jnp.sum(v * v, axis=1)
    uv = jnp.sum(u * v, axis=1)
    alpha = 1.0 - uu
    alpha = jnp.where(alpha <= 0.0, EPS, alpha)
    beta = 1.0 - vv
    beta = jnp.where(beta <= 0.0, EPS, beta)
    gamma = 1.0 + 2.0 * (uu - 2.0 * uv + vv) / alpha / beta
    gamma = jnp.where(gamma < 1.0, 1.0, gamma)
    uu_ref[...] = uu
    uv_ref[...] = uv
    vv_ref[...] = vv
    alpha_ref[...] = alpha
    beta_ref[...] = beta
    gamma_ref[...] = gamma
    dists_ref[...] = jnp.log(gamma + jnp.sqrt(gamma * gamma - 1.0))


_tc_call = pl.pallas_call(
    _tc_body,
    grid=(B // TCB,),
    in_specs=[
        pl.BlockSpec((TCB, 128), lambda i: (i, 0)),
        pl.BlockSpec((TCB, 128), lambda i: (i + B // TCB, 0)),
    ],
    out_specs=[pl.BlockSpec((TCB,), lambda i: (i,))] * 7,
    out_shape=[jax.ShapeDtypeStruct((B,), jnp.float32)] * 7,
)


def kernel(left_idx, right_idx, W):
    stage = _sc_call(W.T, left_idx, right_idx)
    return tuple(_tc_call(stage, stage))


# speedup vs baseline: 3.0604x; 1.4627x over previous
import jax
import jax.numpy as jnp
from jax import lax
from jax.experimental import pallas as pl
from jax.experimental.pallas import tpu as pltpu
from jax.experimental.pallas import tpu_sc as plsc

NC, NS, L = 2, 16, 16
NW = NC * NS
B = 16384
D = 32
V = 1000000
WIN = 1024
NWIN = 977
TAILW = 512
TAILX = 64
KMAX = 31
SCH = 8
NSS = B // (SCH * L)
STAGE_ROWS = 2 * B + NW
EPS = 1e-05


def _sc_body(wt_hbm, left_hbm, right_hbm, stage_hbm,
             idx_v, my_ent, wbuf, ext, grp, tb0, tb1, tb2, tb3,
             cnt_s, sctr_s, dsem0, dsem1, ssem):
    wid = lax.axis_index("s") * NC + lax.axis_index("c")
    lane = lax.iota(jnp.int32, L)
    dlo = lax.iota(jnp.int32, L)
    dhi = dlo + L

    def issue_window(k, issue):
        wb = wid + k * NW
        s = wb * WIN
        for par in range(2):
            @pl.when((wb < NWIN) & (k & 1 == par))
            def _():
                sem = dsem0 if par == 0 else dsem1

                @pl.when(wb < NWIN - 1)
                def _():
                    for db in range(4):
                        cp = pltpu.make_async_copy(
                            wt_hbm.at[pl.ds(db * 8, 8), pl.ds(s, WIN)],
                            wbuf.at[par, pl.ds(db * 8, 8), :], sem)
                        cp.start() if issue else cp.wait()

                @pl.when(wb == NWIN - 1)
                def _():
                    for db in range(4):
                        cp = pltpu.make_async_copy(
                            wt_hbm.at[pl.ds(db * 8, 8), pl.ds(s, TAILW)],
                            wbuf.at[par, pl.ds(db * 8, 8), pl.ds(0, TAILW)],
                            sem)
                        cp.start() if issue else cp.wait()

    def drain_scatter(n):
        def body(_, c):
            pltpu.make_async_copy(ext.at[0], stage_hbm.at[pl.ds(0, L)],
                                  ssem).wait()
            return c
        lax.fori_loop(0, n, body, 0)

    def extract_group(par, gvec):
        cvec = (gvec >> 16) & (WIN - 1)
        slots = gvec & 65535
        g = sctr_s[0]
        epar = lax.rem(g, 4)

        @pl.when(g >= 4)
        def _():
            pltpu.make_async_copy(ext.at[0], stage_hbm.at[pl.ds(0, L)],
                                  ssem).wait()
        for e in range(L):
            col = jnp.full((L,), 0, jnp.int32) + cvec[e]
            lo = plsc.load_gather(wbuf.at[par], [dlo, col])
            hi = plsc.load_gather(wbuf.at[par], [dhi, col])
            ext[epar, e, pl.ds(0, L)] = lo
            ext[epar, e, pl.ds(L, L)] = hi
        pltpu.async_copy(ext.at[epar], stage_hbm.at[slots], ssem)
        sctr_s[0] = g + 1

    def process_window(k):
        cnt = cnt_s[0]
        nss = (cnt + SCH * L - 1) // (SCH * L)
        for par in range(2):
            @pl.when((wid + k * NW < NWIN) & (k & 1 == par))
            def _():
                @pl.when(wid + k * NW == NWIN - 1)
                def _():
                    for db, tb in enumerate((tb0, tb1, tb2, tb3)):
                        pltpu.sync_copy(
                            wt_hbm.at[pl.ds(db * 8, 8),
                                      pl.ds((NWIN - 1) * WIN + TAILW, TAILX)],
                            tb)
                    for db, tb in enumerate((tb0, tb1, tb2, tb3)):
                        for r in range(8):
                            for j in range(TAILX // L):
                                t = tb[r, pl.ds(j * L, L)]
                                wbuf[par, db * 8 + r,
                                     pl.ds(TAILW + j * L, L)] = t

                def ss_body(ss, gtot):
                    base = ss * (SCH * L)
                    ents = []
                    masks = []
                    pv = jnp.zeros((L,), jnp.int32)
                    for t in range(SCH):
                        e_t = my_ent[pl.ds(base + t * L, L)]
                        valid = (base + t * L + lane) < cnt
                        m_t = ((e_t >> 26) == k) & valid
                        pv = jnp.where(
                            lane == t,
                            plsc.all_reduce_population_count(m_t), pv)
                        ents.append(e_t)
                        masks.append(m_t)
                    cs = plsc.cumsum(pv)
                    offs = cs - pv
                    for t in range(SCH):
                        plsc.store_compressed(
                            grp.at[pl.ds(gtot + offs[t], L)],
                            ents[t], mask=masks[t])
                    gtot = gtot + cs[SCH - 1]
                    ngr = gtot >> 4

                    def gext(g, c):
                        extract_group(par, grp[pl.ds(g * L, L)])
                        return c
                    lax.fori_loop(0, ngr, gext, 0)

                    @pl.when(ngr > 0)
                    def _():
                        t = grp[pl.ds(ngr * L, L)]
                        grp[pl.ds(0, L)] = t
                    return gtot & (L - 1)

                gtot = lax.fori_loop(0, nss, ss_body, 0)

                @pl.when(gtot > 0)
                def _():
                    gvec = grp[pl.ds(0, L)]
                    gvec = jnp.where(lane < gtot, gvec, 2 * B + wid)
                    extract_group(par, gvec)

    issue_window(0, True)

    def scan_side(side, idx_hbm, base0):
        pltpu.sync_copy(idx_hbm, idx_v)

        def sscan(ss, base):
            ents = []
            masks = []
            pv = jnp.zeros((L,), jnp.int32)
            for t in range(SCH):
                ks = idx_v[pl.ds((ss * SCH + t) * L, L)]
                m_t = ((ks >> 10) & (NW - 1)) == wid
                slot = (ss * SCH + t) * L + lane + side * B
                ent = ((ks >> 15) << 26) | ((ks & (WIN - 1)) << 16) | slot
                pv = jnp.where(
                    lane == t,
                    plsc.all_reduce_population_count(m_t), pv)
                ents.append(ent)
                masks.append(m_t)
            cs = plsc.cumsum(pv)
            offs = cs - pv
            for t in range(SCH):
                plsc.store_compressed(my_ent.at[pl.ds(base + offs[t], L)],
                                      ents[t], mask=masks[t])
            return base + cs[SCH - 1]

        return lax.fori_loop(0, NSS, sscan, base0)

    cnt = scan_side(0, left_hbm, 0)
    cnt = scan_side(1, right_hbm, cnt)
    cnt_s[0] = cnt
    sctr_s[0] = 0

    def wloop(k, carry):
        issue_window(k + 1, True)
        issue_window(k, False)
        process_window(k)
        return carry

    lax.fori_loop(0, KMAX, wloop, 0)

    g = sctr_s[0]
    drain_scatter(jnp.minimum(g, 4))


_sc_call = pl.kernel(
    _sc_body,
    out_type=jax.ShapeDtypeStruct((STAGE_ROWS, 128), jnp.float32),
    mesh=plsc.VectorSubcoreMesh(
        core_axis_name="c", subcore_axis_name="s",
        num_cores=NC, num_subcores=NS),
    compiler_params=pltpu.CompilerParams(
        needs_layout_passes=False, use_tc_tiling_on_sc=True),
    scratch_types=[
        pltpu.VMEM((B,), jnp.int32),
        pltpu.VMEM((2 * B,), jnp.int32),
        pltpu.VMEM((2, D, WIN), jnp.float32),
        pltpu.VMEM((4, L, 128), jnp.float32),
        pltpu.VMEM((SCH * L + 2 * L,), jnp.int32),
        pltpu.VMEM((8, 64), jnp.float32),
        pltpu.VMEM((8, 64), jnp.float32),
        pltpu.VMEM((8, 64), jnp.float32),
        pltpu.VMEM((8, 64), jnp.float32),
        pltpu.SMEM((1,), jnp.int32),
        pltpu.SMEM((1,), jnp.int32),
        pltpu.SemaphoreType.DMA,
        pltpu.SemaphoreType.DMA,
        pltpu.SemaphoreType.DMA,
    ],
)


TCB = 512


def _tc_body(u_ref, v_ref, uu_ref, uv_ref, vv_ref,
             alpha_ref, beta_ref, gamma_ref, dists_ref):
    u = u_ref[:, 0:D]
    v = v_ref[:, 0:D]
    uu = jnp.sum(u * u, axis=1)
    vv = jnp.sum(v * v, axis=1)
    uv = jnp.sum(u * v, axis=1)
    alpha = 1.0 - uu
    alpha = jnp.where(alpha <= 0.0, EPS, alpha)
    beta = 1.0 - vv
    beta = jnp.where(beta <= 0.0, EPS, beta)
    gamma = 1.0 + 2.0 * (uu - 2.0 * uv + vv) / alpha / beta
    gamma = jnp.where(gamma < 1.0, 1.0, gamma)
    uu_ref[...] = uu
    uv_ref[...] = uv
    vv_ref[...] = vv
    alpha_ref[...] = alpha
    beta_ref[...] = beta
    gamma_ref[...] = gamma
    dists_ref[...] = jnp.log(gamma + jnp.sqrt(gamma * gamma - 1.0))


_tc_call = pl.pallas_call(
    _tc_body,
    grid=(B // TCB,),
    in_specs=[
        pl.BlockSpec((TCB, 128), lambda i: (i, 0)),
        pl.BlockSpec((TCB, 128), lambda i: (i + B // TCB, 0)),
    ],
    out_specs=[pl.BlockSpec((TCB,), lambda i: (i,))] * 7,
    out_shape=[jax.ShapeDtypeStruct((B,), jnp.float32)] * 7,
)


def kernel(left_idx, right_idx, W):
    stage = _sc_call(W.T, left_idx, right_idx)
    return tuple(_tc_call(stage, stage))


# speedup vs baseline: 3.0765x; 1.0053x over previous
import jax
import jax.numpy as jnp
from jax import lax
from jax.experimental import pallas as pl
from jax.experimental.pallas import tpu as pltpu
from jax.experimental.pallas import tpu_sc as plsc

NC, NS, L = 2, 16, 16
NW = NC * NS
B = 16384
D = 32
V = 1000000
WIN = 1024
NWIN = 977
TAILW = 512
TAILX = 64
KMAX = 31
SCH = 8
NSS = B // (SCH * L)
STAGE_ROWS = 2 * B + NW
EPS = 1e-05


def _sc_body(wt_hbm, left_hbm, right_hbm, stage_hbm,
             idx_v, my_ent, wbuf, ext, grp, tb0, tb1, tb2, tb3,
             cnt_s, sctr_s, dsem0, dsem1, ssem):
    wid = lax.axis_index("s") * NC + lax.axis_index("c")
    lane = lax.iota(jnp.int32, L)
    dlo = lax.iota(jnp.int32, L)
    dhi = dlo + L

    def issue_window(k, issue):
        wb = wid + k * NW
        s = wb * WIN
        for par in range(2):
            @pl.when((wb < NWIN) & (k & 1 == par))
            def _():
                sem = dsem0 if par == 0 else dsem1

                @pl.when(wb < NWIN - 1)
                def _():
                    for db in range(4):
                        cp = pltpu.make_async_copy(
                            wt_hbm.at[pl.ds(db * 8, 8), pl.ds(s, WIN)],
                            wbuf.at[par, pl.ds(db * 8, 8), :], sem)
                        cp.start() if issue else cp.wait()

                @pl.when(wb == NWIN - 1)
                def _():
                    for db in range(4):
                        cp = pltpu.make_async_copy(
                            wt_hbm.at[pl.ds(db * 8, 8), pl.ds(s, TAILW)],
                            wbuf.at[par, pl.ds(db * 8, 8), pl.ds(0, TAILW)],
                            sem)
                        cp.start() if issue else cp.wait()

    def drain_scatter(n):
        def body(_, c):
            pltpu.make_async_copy(ext.at[0], stage_hbm.at[pl.ds(0, L)],
                                  ssem).wait()
            return c
        lax.fori_loop(0, n, body, 0)

    def extract_group(par, gvec):
        cvec = (gvec >> 16) & (WIN - 1)
        slots = gvec & 65535
        g = sctr_s[0]
        epar = lax.rem(g, 4)

        @pl.when(g >= 4)
        def _():
            pltpu.make_async_copy(ext.at[0], stage_hbm.at[pl.ds(0, L)],
                                  ssem).wait()
        for e in range(L):
            col = jnp.full((L,), 0, jnp.int32) + cvec[e]
            lo = plsc.load_gather(wbuf.at[par], [dlo, col])
            hi = plsc.load_gather(wbuf.at[par], [dhi, col])
            ext[epar, e, pl.ds(0, L)] = lo
            ext[epar, e, pl.ds(L, L)] = hi
        pltpu.async_copy(ext.at[epar], stage_hbm.at[slots], ssem)
        sctr_s[0] = g + 1

    def process_window(k):
        cnt = cnt_s[0]
        nss = (cnt + SCH * L - 1) // (SCH * L)
        for par in range(2):
            @pl.when((wid + k * NW < NWIN) & (k & 1 == par))
            def _():
                @pl.when(wid + k * NW == NWIN - 1)
                def _():
                    for db, tb in enumerate((tb0, tb1, tb2, tb3)):
                        pltpu.sync_copy(
                            wt_hbm.at[pl.ds(db * 8, 8),
                                      pl.ds((NWIN - 1) * WIN + TAILW, TAILX)],
                            tb)
                    for db, tb in enumerate((tb0, tb1, tb2, tb3)):
                        for r in range(8):
                            for j in range(TAILX // L):
                                t = tb[r, pl.ds(j * L, L)]
                                wbuf[par, db * 8 + r,
                                     pl.ds(TAILW + j * L, L)] = t

                def ss_body(ss, gtot):
                    base = ss * (SCH * L)
                    ents = []
                    masks = []
                    pv = jnp.zeros((L,), jnp.int32)
                    for t in range(SCH):
                        e_t = my_ent[pl.ds(base + t * L, L)]
                        valid = (base + t * L + lane) < cnt
                        m_t = ((e_t >> 26) == k) & valid
                        pv = jnp.where(
                            lane == t,
                            plsc.all_reduce_population_count(m_t), pv)
                        ents.append(e_t)
                        masks.append(m_t)
                    cs = plsc.cumsum(pv)
                    offs = cs - pv
                    for t in range(SCH):
                        plsc.store_compressed(
                            grp.at[pl.ds(gtot + offs[t], L)],
                            ents[t], mask=masks[t])
                    gtot = gtot + cs[SCH - 1]
                    ngr = gtot >> 4

                    def gext(g, c):
                        extract_group(par, grp[pl.ds(g * L, L)])
                        return c
                    lax.fori_loop(0, ngr, gext, 0)

                    @pl.when(ngr > 0)
                    def _():
                        t = grp[pl.ds(ngr * L, L)]
                        grp[pl.ds(0, L)] = t
                    return gtot & (L - 1)

                gtot = lax.fori_loop(0, nss, ss_body, 0)

                @pl.when(gtot > 0)
                def _():
                    gvec = grp[pl.ds(0, L)]
                    gvec = jnp.where(lane < gtot, gvec, 2 * B + wid)
                    extract_group(par, gvec)

    issue_window(0, True)

    def scan_side(side, idx_hbm, base0):
        pltpu.sync_copy(idx_hbm, idx_v)

        def sscan(ss, base):
            ents = []
            masks = []
            pv = jnp.zeros((L,), jnp.int32)
            for t in range(SCH):
                ks = idx_v[pl.ds((ss * SCH + t) * L, L)]
                m_t = ((ks >> 10) & (NW - 1)) == wid
                slot = (ss * SCH + t) * L + lane + side * B
                ent = ((ks >> 15) << 26) | ((ks & (WIN - 1)) << 16) | slot
                pv = jnp.where(
                    lane == t,
                    plsc.all_reduce_population_count(m_t), pv)
                ents.append(ent)
                masks.append(m_t)
            cs = plsc.cumsum(pv)
            offs = cs - pv
            for t in range(SCH):
                plsc.store_compressed(my_ent.at[pl.ds(base + offs[t], L)],
                                      ents[t], mask=masks[t])
            return base + cs[SCH - 1]

        return lax.fori_loop(0, NSS, sscan, base0)

    cnt = scan_side(0, left_hbm, 0)
    cnt = scan_side(1, right_hbm, cnt)
    cnt_s[0] = cnt
    sctr_s[0] = 0

    def wloop(k, carry):
        issue_window(k + 1, True)
        issue_window(k, False)
        process_window(k)
        return carry

    lax.fori_loop(0, KMAX, wloop, 0)

    g = sctr_s[0]
    drain_scatter(jnp.minimum(g, 4))


_sc_call = pl.kernel(
    _sc_body,
    out_type=jax.ShapeDtypeStruct((STAGE_ROWS, 128), jnp.float32),
    mesh=plsc.VectorSubcoreMesh(
        core_axis_name="c", subcore_axis_name="s",
        num_cores=NC, num_subcores=NS),
    compiler_params=pltpu.CompilerParams(
        needs_layout_passes=False, use_tc_tiling_on_sc=True),
    scratch_types=[
        pltpu.VMEM((B,), jnp.int32),
        pltpu.VMEM((2 * B,), jnp.int32),
        pltpu.VMEM((2, D, WIN), jnp.float32),
        pltpu.VMEM((4, L, 128), jnp.float32),
        pltpu.VMEM((SCH * L + 2 * L,), jnp.int32),
        pltpu.VMEM((8, 64), jnp.float32),
        pltpu.VMEM((8, 64), jnp.float32),
        pltpu.VMEM((8, 64), jnp.float32),
        pltpu.VMEM((8, 64), jnp.float32),
        pltpu.SMEM((1,), jnp.int32),
        pltpu.SMEM((1,), jnp.int32),
        pltpu.SemaphoreType.DMA,
        pltpu.SemaphoreType.DMA,
        pltpu.SemaphoreType.DMA,
    ],
)


TCB = 512


def _tc_body(u_ref, v_ref, uu_ref, uv_ref, vv_ref,
             alpha_ref, beta_ref, gamma_ref, dists_ref):
    lanes = lax.broadcasted_iota(jnp.int32, (TCB, 128), 1)
    u = jnp.where(lanes < D, u_ref[...], 0.0)
    v = jnp.where(lanes < D, v_ref[...], 0.0)
    ones = jnp.ones((128,), jnp.float32)
    uu = jnp.dot(u * u, ones)
    vv = jnp.dot(v * v, ones)
    uv = jnp.dot(u * v, ones)
    alpha = 1.0 - uu
    alpha = jnp.where(alpha <= 0.0, EPS, alpha)
    beta = 1.0 - vv
    beta = jnp.where(beta <= 0.0, EPS, beta)
    gamma = 1.0 + 2.0 * (uu - 2.0 * uv + vv) / alpha / beta
    gamma = jnp.where(gamma < 1.0, 1.0, gamma)
    uu_ref[...] = uu
    uv_ref[...] = uv
    vv_ref[...] = vv
    alpha_ref[...] = alpha
    beta_ref[...] = beta
    gamma_ref[...] = gamma
    dists_ref[...] = jnp.log(gamma + jnp.sqrt(gamma * gamma - 1.0))


_tc_call = pl.pallas_call(
    _tc_body,
    grid=(B // TCB,),
    in_specs=[
        pl.BlockSpec((TCB, 128), lambda i: (i, 0)),
        pl.BlockSpec((TCB, 128), lambda i: (i + B // TCB, 0)),
    ],
    out_specs=[pl.BlockSpec((TCB,), lambda i: (i,))] * 7,
    out_shape=[jax.ShapeDtypeStruct((B,), jnp.float32)] * 7,
)


def kernel(left_idx, right_idx, W):
    stage = _sc_call(W.T, left_idx, right_idx)
    return tuple(_tc_call(stage, stage))


# speedup vs baseline: 4.2221x; 1.3724x over previous
import jax
import jax.numpy as jnp
from jax import lax
from jax.experimental import pallas as pl
from jax.experimental.pallas import tpu as pltpu
from jax.experimental.pallas import tpu_sc as plsc

NC, NS, L = 2, 16, 16
NW = NC * NS
B = 16384
D = 32
V = 1000000
WIN = 1024
NWIN = 977
TAILW = 512
TAILX = 64
KMAX = 31
SCH = 8
NSS = B // (SCH * L)
STAGE_ROWS = 2 * B + NW
EPS = 1e-05


def _sc_body(wt_hbm, left_hbm, right_hbm, stage_hbm,
             idx_v, my_ent, wbuf, ext, grp, tb0, tb1, tb2, tb3,
             cnt_s, sctr_s, dsem0, dsem1, ssem):
    wid = lax.axis_index("s") * NC + lax.axis_index("c")
    lane = lax.iota(jnp.int32, L)
    dlo = lax.iota(jnp.int32, L)
    dhi = dlo + L

    def issue_window(k, issue):
        wb = wid + k * NW
        s = wb * WIN
        for par in range(2):
            @pl.when((wb < NWIN) & (k & 1 == par))
            def _():
                sem = dsem0 if par == 0 else dsem1

                @pl.when(wb < NWIN - 1)
                def _():
                    for db in range(4):
                        cp = pltpu.make_async_copy(
                            wt_hbm.at[pl.ds(db * 8, 8), pl.ds(s, WIN)],
                            wbuf.at[par, pl.ds(db * 8, 8), :], sem)
                        cp.start() if issue else cp.wait()

                @pl.when(wb == NWIN - 1)
                def _():
                    for db in range(4):
                        cp = pltpu.make_async_copy(
                            wt_hbm.at[pl.ds(db * 8, 8), pl.ds(s, TAILW)],
                            wbuf.at[par, pl.ds(db * 8, 8), pl.ds(0, TAILW)],
                            sem)
                        cp.start() if issue else cp.wait()

    def drain_scatter(n):
        def body(_, c):
            pltpu.make_async_copy(ext.at[0], stage_hbm.at[pl.ds(0, L)],
                                  ssem).wait()
            return c
        lax.fori_loop(0, n, body, 0)

    def extract_group(par, gvec):
        cvec = (gvec >> 16) & (WIN - 1)
        slots = gvec & 65535
        g = sctr_s[0]
        epar = lax.rem(g, 4)

        @pl.when(g >= 4)
        def _():
            pltpu.make_async_copy(ext.at[0], stage_hbm.at[pl.ds(0, L)],
                                  ssem).wait()
        for e in range(L):
            col = jnp.full((L,), 0, jnp.int32) + cvec[e]
            lo = plsc.load_gather(wbuf.at[par], [dlo, col])
            hi = plsc.load_gather(wbuf.at[par], [dhi, col])
            ext[epar, e, pl.ds(0, L)] = lo
            ext[epar, e, pl.ds(L, L)] = hi
        pltpu.async_copy(ext.at[epar], stage_hbm.at[slots], ssem)
        sctr_s[0] = g + 1

    def process_window(k):
        cnt = cnt_s[0]
        nss = (cnt + SCH * L - 1) // (SCH * L)
        for par in range(2):
            @pl.when((wid + k * NW < NWIN) & (k & 1 == par))
            def _():
                @pl.when(wid + k * NW == NWIN - 1)
                def _():
                    for db, tb in enumerate((tb0, tb1, tb2, tb3)):
                        pltpu.sync_copy(
                            wt_hbm.at[pl.ds(db * 8, 8),
                                      pl.ds((NWIN - 1) * WIN + TAILW, TAILX)],
                            tb)
                    for db, tb in enumerate((tb0, tb1, tb2, tb3)):
                        for r in range(8):
                            for j in range(TAILX // L):
                                t = tb[r, pl.ds(j * L, L)]
                                wbuf[par, db * 8 + r,
                                     pl.ds(TAILW + j * L, L)] = t

                def ss_body(ss, gtot):
                    base = ss * (SCH * L)
                    ents = []
                    masks = []
                    pv = jnp.zeros((L,), jnp.int32)
                    for t in range(SCH):
                        e_t = my_ent[pl.ds(base + t * L, L)]
                        valid = (base + t * L + lane) < cnt
                        m_t = ((e_t >> 26) == k) & valid
                        pv = jnp.where(
                            lane == t,
                            plsc.all_reduce_population_count(m_t), pv)
                        ents.append(e_t)
                        masks.append(m_t)
                    cs = plsc.cumsum(pv)
                    offs = cs - pv
                    for t in range(SCH):
                        plsc.store_compressed(
                            grp.at[pl.ds(gtot + offs[t], L)],
                            ents[t], mask=masks[t])
                    gtot = gtot + cs[SCH - 1]
                    ngr = gtot >> 4

                    def gext(g, c):
                        extract_group(par, grp[pl.ds(g * L, L)])
                        return c
                    lax.fori_loop(0, ngr, gext, 0)

                    @pl.when(ngr > 0)
                    def _():
                        t = grp[pl.ds(ngr * L, L)]
                        grp[pl.ds(0, L)] = t
                    return gtot & (L - 1)

                gtot = lax.fori_loop(0, nss, ss_body, 0)

                @pl.when(gtot > 0)
                def _():
                    gvec = grp[pl.ds(0, L)]
                    gvec = jnp.where(lane < gtot, gvec, 2 * B + wid)
                    extract_group(par, gvec)

    issue_window(0, True)

    def scan_side(side, idx_hbm, base0):
        pltpu.sync_copy(idx_hbm, idx_v)

        def sscan(ss, base):
            ents = []
            masks = []
            pv = jnp.zeros((L,), jnp.int32)
            for t in range(SCH):
                ks = idx_v[pl.ds((ss * SCH + t) * L, L)]
                m_t = ((ks >> 10) & (NW - 1)) == wid
                slot = (ss * SCH + t) * L + lane + side * B
                ent = ((ks >> 15) << 26) | ((ks & (WIN - 1)) << 16) | slot
                pv = jnp.where(
                    lane == t,
                    plsc.all_reduce_population_count(m_t), pv)
                ents.append(ent)
                masks.append(m_t)
            cs = plsc.cumsum(pv)
            offs = cs - pv
            for t in range(SCH):
                plsc.store_compressed(my_ent.at[pl.ds(base + offs[t], L)],
                                      ents[t], mask=masks[t])
            return base + cs[SCH - 1]

        return lax.fori_loop(0, NSS, sscan, base0)

    cnt = scan_side(0, left_hbm, 0)
    cnt = scan_side(1, right_hbm, cnt)
    cnt_s[0] = cnt
    sctr_s[0] = 0

    def wloop(k, carry):
        issue_window(k + 1, True)
        issue_window(k, False)
        process_window(k)
        return carry

    lax.fori_loop(0, KMAX, wloop, 0)

    g = sctr_s[0]
    drain_scatter(jnp.minimum(g, 4))


_sc_call = pl.kernel(
    _sc_body,
    out_type=jax.ShapeDtypeStruct((STAGE_ROWS, 128), jnp.float32),
    mesh=plsc.VectorSubcoreMesh(
        core_axis_name="c", subcore_axis_name="s",
        num_cores=NC, num_subcores=NS),
    compiler_params=pltpu.CompilerParams(
        needs_layout_passes=False, use_tc_tiling_on_sc=True),
    scratch_types=[
        pltpu.VMEM((B,), jnp.int32),
        pltpu.VMEM((2 * B,), jnp.int32),
        pltpu.VMEM((2, D, WIN), jnp.float32),
        pltpu.VMEM((4, L, 128), jnp.float32),
        pltpu.VMEM((SCH * L + 2 * L,), jnp.int32),
        pltpu.VMEM((8, 64), jnp.float32),
        pltpu.VMEM((8, 64), jnp.float32),
        pltpu.VMEM((8, 64), jnp.float32),
        pltpu.VMEM((8, 64), jnp.float32),
        pltpu.SMEM((1,), jnp.int32),
        pltpu.SMEM((1,), jnp.int32),
        pltpu.SemaphoreType.DMA,
        pltpu.SemaphoreType.DMA,
        pltpu.SemaphoreType.DMA,
    ],
)


BPW = B // NW


def _dots_body(stage_hbm, uu_hbm, uv_hbm, vv_hbm, u_v, v_v,
               uu_v, uv_v, vv_v):
    wid = lax.axis_index("s") * NC + lax.axis_index("c")
    base = wid * BPW
    lane = lax.iota(jnp.int32, L)
    HB = BPW // 2

    for h in range(2):
        hb = base + h * HB
        pltpu.sync_copy(stage_hbm.at[pl.ds(hb, HB), :], u_v)
        pltpu.sync_copy(stage_hbm.at[pl.ds(B + hb, HB), :], v_v)

        def group(g, carry):
            r0 = g * L
            uu = jnp.zeros((L,), jnp.float32)
            vv = jnp.zeros((L,), jnp.float32)
            uv = jnp.zeros((L,), jnp.float32)
            for k in range(L):
                u0 = u_v[r0 + k, pl.ds(0, L)]
                u1 = u_v[r0 + k, pl.ds(L, L)]
                v0 = v_v[r0 + k, pl.ds(0, L)]
                v1 = v_v[r0 + k, pl.ds(L, L)]
                m = lane == k
                uu = jnp.where(m, jnp.sum(u0 * u0 + u1 * u1), uu)
                vv = jnp.where(m, jnp.sum(v0 * v0 + v1 * v1), vv)
                uv = jnp.where(m, jnp.sum(u0 * v0 + u1 * v1), uv)
            uu_v[pl.ds(h * HB + r0, L)] = uu
            vv_v[pl.ds(h * HB + r0, L)] = vv
            uv_v[pl.ds(h * HB + r0, L)] = uv
            return carry

        lax.fori_loop(0, HB // L, group, 0)
    pltpu.sync_copy(uu_v, uu_hbm.at[pl.ds(base, BPW)])
    pltpu.sync_copy(uv_v, uv_hbm.at[pl.ds(base, BPW)])
    pltpu.sync_copy(vv_v, vv_hbm.at[pl.ds(base, BPW)])


_dots_call = pl.kernel(
    _dots_body,
    out_type=[jax.ShapeDtypeStruct((B,), jnp.float32)] * 3,
    mesh=plsc.VectorSubcoreMesh(
        core_axis_name="c", subcore_axis_name="s",
        num_cores=NC, num_subcores=NS),
    compiler_params=pltpu.CompilerParams(
        needs_layout_passes=False, use_tc_tiling_on_sc=True),
    scratch_types=[
        pltpu.VMEM((BPW // 2, 128), jnp.float32),
        pltpu.VMEM((BPW // 2, 128), jnp.float32),
        pltpu.VMEM((BPW,), jnp.float32),
        pltpu.VMEM((BPW,), jnp.float32),
        pltpu.VMEM((BPW,), jnp.float32),
    ],
)


def _tc_body(uu_ref, uv_ref, vv_ref,
             alpha_ref, beta_ref, gamma_ref, dists_ref):
    uu = uu_ref[...]
    uv = uv_ref[...]
    vv = vv_ref[...]
    alpha = 1.0 - uu
    alpha = jnp.where(alpha <= 0.0, EPS, alpha)
    beta = 1.0 - vv
    beta = jnp.where(beta <= 0.0, EPS, beta)
    gamma = 1.0 + 2.0 * (uu - 2.0 * uv + vv) / alpha / beta
    gamma = jnp.where(gamma < 1.0, 1.0, gamma)
    alpha_ref[...] = alpha
    beta_ref[...] = beta
    gamma_ref[...] = gamma
    dists_ref[...] = jnp.log(gamma + jnp.sqrt(gamma * gamma - 1.0))


_tc_call = pl.pallas_call(
    _tc_body,
    out_shape=[jax.ShapeDtypeStruct((B,), jnp.float32)] * 4,
)


def kernel(left_idx, right_idx, W):
    stage = _sc_call(W.T, left_idx, right_idx)
    uu, uv, vv = _dots_call(stage)
    alpha, beta, gamma, dists = _tc_call(uu, uv, vv)
    return (uu, uv, vv, alpha, beta, gamma, dists)


# speedup vs baseline: 4.2711x; 1.0116x over previous
import jax
import jax.numpy as jnp
from jax import lax
from jax.experimental import pallas as pl
from jax.experimental.pallas import tpu as pltpu
from jax.experimental.pallas import tpu_sc as plsc

NC, NS, L = 2, 16, 16
NW = NC * NS
B = 16384
D = 32
V = 1000000
WIN = 1024
NWIN = 977
TAILW = 512
TAILX = 64
KMAX = 31
SCH = 8
NSS = B // (SCH * L)
STAGE_ROWS = 2 * B + NW
EPS = 1e-05


def _sc_body(wt_hbm, left_hbm, right_hbm, stage_hbm,
             idx_v, my_ent, wbuf, ext, ext2, grp, tb0, tb1, tb2, tb3,
             cnt_s, sctr_s, dsem0, dsem1, ssem):
    wid = lax.axis_index("s") * NC + lax.axis_index("c")
    lane = lax.iota(jnp.int32, L)
    dlo = lax.iota(jnp.int32, L)
    dhi = dlo + L

    def issue_window(k, issue):
        wb = wid + k * NW
        s = wb * WIN
        for par in range(2):
            @pl.when((wb < NWIN) & (k & 1 == par))
            def _():
                sem = dsem0 if par == 0 else dsem1

                @pl.when(wb < NWIN - 1)
                def _():
                    for db in range(4):
                        cp = pltpu.make_async_copy(
                            wt_hbm.at[pl.ds(db * 8, 8), pl.ds(s, WIN)],
                            wbuf.at[par, pl.ds(db * 8, 8), :], sem)
                        cp.start() if issue else cp.wait()

                @pl.when(wb == NWIN - 1)
                def _():
                    for db in range(4):
                        cp = pltpu.make_async_copy(
                            wt_hbm.at[pl.ds(db * 8, 8), pl.ds(s, TAILW)],
                            wbuf.at[par, pl.ds(db * 8, 8), pl.ds(0, TAILW)],
                            sem)
                        cp.start() if issue else cp.wait()

    def drain_scatter(n):
        def body(_, c):
            pltpu.make_async_copy(ext.at[0], stage_hbm.at[pl.ds(0, L)],
                                  ssem).wait()
            return c
        lax.fori_loop(0, n, body, 0)

    def extract_group(par, gvec):
        cvec = (gvec >> 16) & (WIN - 1)
        slots = gvec & 65535
        g = sctr_s[0]
        epar = lax.rem(g, 4)

        @pl.when(g >= 4)
        def _():
            pltpu.make_async_copy(ext.at[0], stage_hbm.at[pl.ds(0, L)],
                                  ssem).wait()
        for d in range(D):
            row = jnp.full((L,), d, jnp.int32)
            vec = plsc.load_gather(wbuf.at[par], [row, cvec])
            ext2[pl.ds(d * 17, L)] = vec
        d17 = dlo * 17
        for e in range(L):
            lo = plsc.load_gather(ext2, [d17 + e])
            hi = plsc.load_gather(ext2, [d17 + (L * 17 + e)])
            ext[epar, e, pl.ds(0, L)] = lo
            ext[epar, e, pl.ds(L, L)] = hi
        pltpu.async_copy(ext.at[epar], stage_hbm.at[slots], ssem)
        sctr_s[0] = g + 1

    def process_window(k):
        cnt = cnt_s[0]
        nss = (cnt + SCH * L - 1) // (SCH * L)
        for par in range(2):
            @pl.when((wid + k * NW < NWIN) & (k & 1 == par))
            def _():
                @pl.when(wid + k * NW == NWIN - 1)
                def _():
                    for db, tb in enumerate((tb0, tb1, tb2, tb3)):
                        pltpu.sync_copy(
                            wt_hbm.at[pl.ds(db * 8, 8),
                                      pl.ds((NWIN - 1) * WIN + TAILW, TAILX)],
                            tb)
                    for db, tb in enumerate((tb0, tb1, tb2, tb3)):
                        for r in range(8):
                            for j in range(TAILX // L):
                                t = tb[r, pl.ds(j * L, L)]
                                wbuf[par, db * 8 + r,
                                     pl.ds(TAILW + j * L, L)] = t

                def ss_body(ss, gtot):
                    base = ss * (SCH * L)
                    ents = []
                    masks = []
                    pv = jnp.zeros((L,), jnp.int32)
                    for t in range(SCH):
                        e_t = my_ent[pl.ds(base + t * L, L)]
                        valid = (base + t * L + lane) < cnt
                        m_t = ((e_t >> 26) == k) & valid
                        pv = jnp.where(
                            lane == t,
                            plsc.all_reduce_population_count(m_t), pv)
                        ents.append(e_t)
                        masks.append(m_t)
                    cs = plsc.cumsum(pv)
                    offs = cs - pv
                    for t in range(SCH):
                        plsc.store_compressed(
                            grp.at[pl.ds(gtot + offs[t], L)],
                            ents[t], mask=masks[t])
                    gtot = gtot + cs[SCH - 1]
                    ngr = gtot >> 4

                    def gext(g, c):
                        extract_group(par, grp[pl.ds(g * L, L)])
                        return c
                    lax.fori_loop(0, ngr, gext, 0)

                    @pl.when(ngr > 0)
                    def _():
                        t = grp[pl.ds(ngr * L, L)]
                        grp[pl.ds(0, L)] = t
                    return gtot & (L - 1)

                gtot = lax.fori_loop(0, nss, ss_body, 0)

                @pl.when(gtot > 0)
                def _():
                    gvec = grp[pl.ds(0, L)]
                    gvec = jnp.where(lane < gtot, gvec, 2 * B + wid)
                    extract_group(par, gvec)

    issue_window(0, True)

    def scan_side(side, idx_hbm, base0):
        pltpu.sync_copy(idx_hbm, idx_v)

        def sscan(ss, base):
            ents = []
            masks = []
            pv = jnp.zeros((L,), jnp.int32)
            for t in range(SCH):
                ks = idx_v[pl.ds((ss * SCH + t) * L, L)]
                m_t = ((ks >> 10) & (NW - 1)) == wid
                slot = (ss * SCH + t) * L + lane + side * B
                ent = ((ks >> 15) << 26) | ((ks & (WIN - 1)) << 16) | slot
                pv = jnp.where(
                    lane == t,
                    plsc.all_reduce_population_count(m_t), pv)
                ents.append(ent)
                masks.append(m_t)
            cs = plsc.cumsum(pv)
            offs = cs - pv
            for t in range(SCH):
                plsc.store_compressed(my_ent.at[pl.ds(base + offs[t], L)],
                                      ents[t], mask=masks[t])
            return base + cs[SCH - 1]

        return lax.fori_loop(0, NSS, sscan, base0)

    cnt = scan_side(0, left_hbm, 0)
    cnt = scan_side(1, right_hbm, cnt)
    cnt_s[0] = cnt
    sctr_s[0] = 0

    def wloop(k, carry):
        issue_window(k + 1, True)
        issue_window(k, False)
        process_window(k)
        return carry

    lax.fori_loop(0, KMAX, wloop, 0)

    g = sctr_s[0]
    drain_scatter(jnp.minimum(g, 4))


_sc_call = pl.kernel(
    _sc_body,
    out_type=jax.ShapeDtypeStruct((STAGE_ROWS, 128), jnp.float32),
    mesh=plsc.VectorSubcoreMesh(
        core_axis_name="c", subcore_axis_name="s",
        num_cores=NC, num_subcores=NS),
    compiler_params=pltpu.CompilerParams(
        needs_layout_passes=False, use_tc_tiling_on_sc=True),
    scratch_types=[
        pltpu.VMEM((B,), jnp.int32),
        pltpu.VMEM((2 * B,), jnp.int32),
        pltpu.VMEM((2, D, WIN), jnp.float32),
        pltpu.VMEM((4, L, 128), jnp.float32),
        pltpu.VMEM((17 * D,), jnp.float32),
        pltpu.VMEM((SCH * L + 2 * L,), jnp.int32),
        pltpu.VMEM((8, 64), jnp.float32),
        pltpu.VMEM((8, 64), jnp.float32),
        pltpu.VMEM((8, 64), jnp.float32),
        pltpu.VMEM((8, 64), jnp.float32),
        pltpu.SMEM((1,), jnp.int32),
        pltpu.SMEM((1,), jnp.int32),
        pltpu.SemaphoreType.DMA,
        pltpu.SemaphoreType.DMA,
        pltpu.SemaphoreType.DMA,
    ],
)


BPW = B // NW


def _dots_body(stage_hbm, uu_hbm, uv_hbm, vv_hbm, u_v, v_v,
               uu_v, uv_v, vv_v):
    wid = lax.axis_index("s") * NC + lax.axis_index("c")
    base = wid * BPW
    lane = lax.iota(jnp.int32, L)
    HB = BPW // 2

    for h in range(2):
        hb = base + h * HB
        pltpu.sync_copy(stage_hbm.at[pl.ds(hb, HB), :], u_v)
        pltpu.sync_copy(stage_hbm.at[pl.ds(B + hb, HB), :], v_v)

        def group(g, carry):
            r0 = g * L
            uu = jnp.zeros((L,), jnp.float32)
            vv = jnp.zeros((L,), jnp.float32)
            uv = jnp.zeros((L,), jnp.float32)
            for k in range(L):
                u0 = u_v[r0 + k, pl.ds(0, L)]
                u1 = u_v[r0 + k, pl.ds(L, L)]
                v0 = v_v[r0 + k, pl.ds(0, L)]
                v1 = v_v[r0 + k, pl.ds(L, L)]
                m = lane == k
                uu = jnp.where(m, jnp.sum(u0 * u0 + u1 * u1), uu)
                vv = jnp.where(m, jnp.sum(v0 * v0 + v1 * v1), vv)
                uv = jnp.where(m, jnp.sum(u0 * v0 + u1 * v1), uv)
            uu_v[pl.ds(h * HB + r0, L)] = uu
            vv_v[pl.ds(h * HB + r0, L)] = vv
            uv_v[pl.ds(h * HB + r0, L)] = uv
            return carry

        lax.fori_loop(0, HB // L, group, 0)
    pltpu.sync_copy(uu_v, uu_hbm.at[pl.ds(base, BPW)])
    pltpu.sync_copy(uv_v, uv_hbm.at[pl.ds(base, BPW)])
    pltpu.sync_copy(vv_v, vv_hbm.at[pl.ds(base, BPW)])


_dots_call = pl.kernel(
    _dots_body,
    out_type=[jax.ShapeDtypeStruct((B,), jnp.float32)] * 3,
    mesh=plsc.VectorSubcoreMesh(
        core_axis_name="c", subcore_axis_name="s",
        num_cores=NC, num_subcores=NS),
    compiler_params=pltpu.CompilerParams(
        needs_layout_passes=False, use_tc_tiling_on_sc=True),
    scratch_types=[
        pltpu.VMEM((BPW // 2, 128), jnp.float32),
        pltpu.VMEM((BPW // 2, 128), jnp.float32),
        pltpu.VMEM((BPW,), jnp.float32),
        pltpu.VMEM((BPW,), jnp.float32),
        pltpu.VMEM((BPW,), jnp.float32),
    ],
)


def _tc_body(uu_ref, uv_ref, vv_ref,
             alpha_ref, beta_ref, gamma_ref, dists_ref):
    uu = uu_ref[...]
    uv = uv_ref[...]
    vv = vv_ref[...]
    alpha = 1.0 - uu
    alpha = jnp.where(alpha <= 0.0, EPS, alpha)
    beta = 1.0 - vv
    beta = jnp.where(beta <= 0.0, EPS, beta)
    gamma = 1.0 + 2.0 * (uu - 2.0 * uv + vv) / alpha / beta
    gamma = jnp.where(gamma < 1.0, 1.0, gamma)
    alpha_ref[...] = alpha
    beta_ref[...] = beta
    gamma_ref[...] = gamma
    dists_ref[...] = jnp.log(gamma + jnp.sqrt(gamma * gamma - 1.0))


_tc_call = pl.pallas_call(
    _tc_body,
    out_shape=[jax.ShapeDtypeStruct((B,), jnp.float32)] * 4,
)


def kernel(left_idx, right_idx, W):
    stage = _sc_call(W.T, left_idx, right_idx)
    uu, uv, vv = _dots_call(stage)
    alpha, beta, gamma, dists = _tc_call(uu, uv, vv)
    return (uu, uv, vv, alpha, beta, gamma, dists)


# speedup vs baseline: 4.3036x; 1.0076x over previous
import jax
import jax.numpy as jnp
from jax import lax
from jax.experimental import pallas as pl
from jax.experimental.pallas import tpu as pltpu
from jax.experimental.pallas import tpu_sc as plsc

NC, NS, L = 2, 16, 16
NW = NC * NS
B = 16384
D = 32
V = 1000000
WIN = 1024
NWIN = 977
TAILW = 512
TAILX = 64
KMAX = 31
SCH = 8
NSS = B // (SCH * L)
STAGE_ROWS = 2 * B + NW
EPS = 1e-05


def _sc_body(wt_hbm, left_hbm, right_hbm, stage_hbm,
             idx_v, my_ent, wbuf, ext, ext2, grp, tb0, tb1, tb2, tb3,
             cnt_s, sctr_s, dsem0, dsem1, ssem):
    wid = lax.axis_index("s") * NC + lax.axis_index("c")
    lane = lax.iota(jnp.int32, L)
    dlo = lax.iota(jnp.int32, L)
    dhi = dlo + L

    def issue_window(k, issue):
        wb = wid + k * NW
        s = wb * WIN
        for par in range(2):
            @pl.when((wb < NWIN) & (k & 1 == par))
            def _():
                sem = dsem0 if par == 0 else dsem1

                @pl.when(wb < NWIN - 1)
                def _():
                    for db in range(4):
                        cp = pltpu.make_async_copy(
                            wt_hbm.at[pl.ds(db * 8, 8), pl.ds(s, WIN)],
                            wbuf.at[par, pl.ds(db * 8, 8), :], sem)
                        cp.start() if issue else cp.wait()

                @pl.when(wb == NWIN - 1)
                def _():
                    for db in range(4):
                        cp = pltpu.make_async_copy(
                            wt_hbm.at[pl.ds(db * 8, 8), pl.ds(s, TAILW)],
                            wbuf.at[par, pl.ds(db * 8, 8), pl.ds(0, TAILW)],
                            sem)
                        cp.start() if issue else cp.wait()

    def drain_scatter(n):
        def body(_, c):
            pltpu.make_async_copy(ext.at[0], stage_hbm.at[pl.ds(0, L)],
                                  ssem).wait()
            return c
        lax.fori_loop(0, n, body, 0)

    def extract_group(par, gvec):
        cvec = (gvec >> 16) & (WIN - 1)
        slots = gvec & 65535
        g = sctr_s[0]
        epar = lax.rem(g, 4)

        @pl.when(g >= 4)
        def _():
            pltpu.make_async_copy(ext.at[0], stage_hbm.at[pl.ds(0, L)],
                                  ssem).wait()
        for d in range(D):
            row = jnp.full((L,), d, jnp.int32)
            vec = plsc.load_gather(wbuf.at[par], [row, cvec])
            ext2[pl.ds(d * 17, L)] = vec
        d17 = dlo * 17
        for e in range(L):
            lo = plsc.load_gather(ext2, [d17 + e])
            hi = plsc.load_gather(ext2, [d17 + (L * 17 + e)])
            ext[epar, e, pl.ds(0, L)] = lo
            ext[epar, e, pl.ds(L, L)] = hi
        pltpu.async_copy(ext.at[epar], stage_hbm.at[slots], ssem)
        sctr_s[0] = g + 1

    def process_window(k):
        cnt = cnt_s[0]
        nss = (cnt + SCH * L - 1) // (SCH * L)
        for par in range(2):
            @pl.when((wid + k * NW < NWIN) & (k & 1 == par))
            def _():
                @pl.when(wid + k * NW == NWIN - 1)
                def _():
                    for db, tb in enumerate((tb0, tb1, tb2, tb3)):
                        pltpu.sync_copy(
                            wt_hbm.at[pl.ds(db * 8, 8),
                                      pl.ds((NWIN - 1) * WIN + TAILW, TAILX)],
                            tb)
                    for db, tb in enumerate((tb0, tb1, tb2, tb3)):
                        for r in range(8):
                            for j in range(TAILX // L):
                                t = tb[r, pl.ds(j * L, L)]
                                wbuf[par, db * 8 + r,
                                     pl.ds(TAILW + j * L, L)] = t

                def ss_body(ss, gtot):
                    base = ss * (SCH * L)
                    ents = []
                    masks = []
                    pv = jnp.zeros((L,), jnp.int32)
                    for t in range(SCH):
                        e_t = my_ent[pl.ds(base + t * L, L)]
                        valid = (base + t * L + lane) < cnt
                        m_t = ((e_t >> 26) == k) & valid
                        pv = jnp.where(
                            lane == t,
                            plsc.all_reduce_population_count(m_t), pv)
                        ents.append(e_t)
                        masks.append(m_t)
                    cs = plsc.cumsum(pv)
                    offs = cs - pv
                    for t in range(SCH):
                        plsc.store_compressed(
                            grp.at[pl.ds(gtot + offs[t], L)],
                            ents[t], mask=masks[t])
                    gtot = gtot + cs[SCH - 1]
                    ngr = gtot >> 4

                    def gext(g, c):
                        extract_group(par, grp[pl.ds(g * L, L)])
                        return c
                    lax.fori_loop(0, ngr, gext, 0)

                    @pl.when(ngr > 0)
                    def _():
                        t = grp[pl.ds(ngr * L, L)]
                        grp[pl.ds(0, L)] = t
                    return gtot & (L - 1)

                gtot = lax.fori_loop(0, nss, ss_body, 0)

                @pl.when(gtot > 0)
                def _():
                    gvec = grp[pl.ds(0, L)]
                    gvec = jnp.where(lane < gtot, gvec, 2 * B + wid)
                    extract_group(par, gvec)

    issue_window(0, True)

    def scan_side(side, idx_hbm, base0):
        pltpu.sync_copy(idx_hbm, idx_v)

        def sscan(ss, base):
            ents = []
            masks = []
            pv = jnp.zeros((L,), jnp.int32)
            for t in range(SCH):
                ks = idx_v[pl.ds((ss * SCH + t) * L, L)]
                m_t = ((ks >> 10) & (NW - 1)) == wid
                slot = (ss * SCH + t) * L + lane + side * B
                ent = ((ks >> 15) << 26) | ((ks & (WIN - 1)) << 16) | slot
                pv = jnp.where(
                    lane == t,
                    plsc.all_reduce_population_count(m_t), pv)
                ents.append(ent)
                masks.append(m_t)
            cs = plsc.cumsum(pv)
            offs = cs - pv
            for t in range(SCH):
                plsc.store_compressed(my_ent.at[pl.ds(base + offs[t], L)],
                                      ents[t], mask=masks[t])
            return base + cs[SCH - 1]

        return lax.fori_loop(0, NSS, sscan, base0)

    cnt = scan_side(0, left_hbm, 0)
    cnt = scan_side(1, right_hbm, cnt)
    cnt_s[0] = cnt
    sctr_s[0] = 0

    def wloop(k, carry):
        issue_window(k + 1, True)
        issue_window(k, False)
        process_window(k)
        return carry

    lax.fori_loop(0, KMAX, wloop, 0)

    g = sctr_s[0]
    drain_scatter(jnp.minimum(g, 4))


_sc_call = pl.kernel(
    _sc_body,
    out_type=jax.ShapeDtypeStruct((STAGE_ROWS, 128), jnp.float32),
    mesh=plsc.VectorSubcoreMesh(
        core_axis_name="c", subcore_axis_name="s",
        num_cores=NC, num_subcores=NS),
    compiler_params=pltpu.CompilerParams(
        needs_layout_passes=False, use_tc_tiling_on_sc=True),
    scratch_types=[
        pltpu.VMEM((B,), jnp.int32),
        pltpu.VMEM((2 * B,), jnp.int32),
        pltpu.VMEM((2, D, WIN), jnp.float32),
        pltpu.VMEM((4, L, 128), jnp.float32),
        pltpu.VMEM((17 * D,), jnp.float32),
        pltpu.VMEM((SCH * L + 2 * L,), jnp.int32),
        pltpu.VMEM((8, 64), jnp.float32),
        pltpu.VMEM((8, 64), jnp.float32),
        pltpu.VMEM((8, 64), jnp.float32),
        pltpu.VMEM((8, 64), jnp.float32),
        pltpu.SMEM((1,), jnp.int32),
        pltpu.SMEM((1,), jnp.int32),
        pltpu.SemaphoreType.DMA,
        pltpu.SemaphoreType.DMA,
        pltpu.SemaphoreType.DMA,
    ],
)


BPW = B // NW


def _dots_body(stage_hbm, uu_hbm, uv_hbm, vv_hbm, u_v, v_v,
               uu_v, uv_v, vv_v, dsem):
    wid = lax.axis_index("s") * NC + lax.axis_index("c")
    base = wid * BPW
    lane = lax.iota(jnp.int32, L)
    HB = BPW // 2

    for h in range(2):
        hb = base + h * HB
        c1 = pltpu.async_copy(stage_hbm.at[pl.ds(hb, HB), :], u_v, dsem)
        c2 = pltpu.async_copy(stage_hbm.at[pl.ds(B + hb, HB), :], v_v, dsem)
        c1.wait()
        c2.wait()

        def group(g, carry):
            r0 = g * L
            uu = jnp.zeros((L,), jnp.float32)
            vv = jnp.zeros((L,), jnp.float32)
            uv = jnp.zeros((L,), jnp.float32)
            for k in range(L):
                u0 = u_v[r0 + k, pl.ds(0, L)]
                u1 = u_v[r0 + k, pl.ds(L, L)]
                v0 = v_v[r0 + k, pl.ds(0, L)]
                v1 = v_v[r0 + k, pl.ds(L, L)]
                m = lane == k
                uu = jnp.where(m, jnp.sum(u0 * u0 + u1 * u1), uu)
                vv = jnp.where(m, jnp.sum(v0 * v0 + v1 * v1), vv)
                uv = jnp.where(m, jnp.sum(u0 * v0 + u1 * v1), uv)
            uu_v[pl.ds(h * HB + r0, L)] = uu
            vv_v[pl.ds(h * HB + r0, L)] = vv
            uv_v[pl.ds(h * HB + r0, L)] = uv
            return carry

        lax.fori_loop(0, HB // L, group, 0)
    pltpu.sync_copy(uu_v, uu_hbm.at[pl.ds(base, BPW)])
    pltpu.sync_copy(uv_v, uv_hbm.at[pl.ds(base, BPW)])
    pltpu.sync_copy(vv_v, vv_hbm.at[pl.ds(base, BPW)])


_dots_call = pl.kernel(
    _dots_body,
    out_type=[jax.ShapeDtypeStruct((B,), jnp.float32)] * 3,
    mesh=plsc.VectorSubcoreMesh(
        core_axis_name="c", subcore_axis_name="s",
        num_cores=NC, num_subcores=NS),
    compiler_params=pltpu.CompilerParams(
        needs_layout_passes=False, use_tc_tiling_on_sc=True),
    scratch_types=[
        pltpu.VMEM((BPW // 2, 128), jnp.float32),
        pltpu.VMEM((BPW // 2, 128), jnp.float32),
        pltpu.VMEM((BPW,), jnp.float32),
        pltpu.VMEM((BPW,), jnp.float32),
        pltpu.VMEM((BPW,), jnp.float32),
        pltpu.SemaphoreType.DMA,
    ],
)


def _tc_body(uu_ref, uv_ref, vv_ref,
             alpha_ref, beta_ref, gamma_ref, dists_ref):
    uu = uu_ref[...]
    uv = uv_ref[...]
    vv = vv_ref[...]
    alpha = 1.0 - uu
    alpha = jnp.where(alpha <= 0.0, EPS, alpha)
    beta = 1.0 - vv
    beta = jnp.where(beta <= 0.0, EPS, beta)
    gamma = 1.0 + 2.0 * (uu - 2.0 * uv + vv) / alpha / beta
    gamma = jnp.where(gamma < 1.0, 1.0, gamma)
    alpha_ref[...] = alpha
    beta_ref[...] = beta
    gamma_ref[...] = gamma
    dists_ref[...] = jnp.log(gamma + jnp.sqrt(gamma * gamma - 1.0))


_tc_call = pl.pallas_call(
    _tc_body,
    out_shape=[jax.ShapeDtypeStruct((B,), jnp.float32)] * 4,
)


def kernel(left_idx, right_idx, W):
    stage = _sc_call(W.T, left_idx, right_idx)
    uu, uv, vv = _dots_call(stage)
    alpha, beta, gamma, dists = _tc_call(uu, uv, vv)
    return (uu, uv, vv, alpha, beta, gamma, dists)
